# baseline (speedup 1.0000x reference)
import functools

import jax
import jax.numpy as jnp
from jax import lax
from jax.experimental import pallas as pl
from jax.experimental.pallas import tpu as pltpu
from jax.experimental.pallas import tpu_sc as plsc

N = 10000
E = 320000
D = 128
H = 256
C = 10

NPAD = 10112
RB = NPAD // 16

NC = 2
NS = 16

KD = 80
DROWS = E // KD
K = 40
ECH = E // (K * NC * NS)
NBUF = 5


def _f32(shape):
  return jax.ShapeDtypeStruct(shape, jnp.float32)


@functools.lru_cache(maxsize=None)
def _sc_mesh():
  return plsc.VectorSubcoreMesh(
      core_axis_name="c", subcore_axis_name="s", num_cores=NC, num_subcores=NS)


DEGW = 8


def _sc_degree_body(dst_hbm, zeros_hbm, ones_hbm, out_hbm, dstv, onesv, acc):
  c = lax.axis_index("c")
  s = lax.axis_index("s")
  nch = DROWS // (NC * NS)
  row0 = s * RB
  pltpu.sync_copy(zeros_hbm.at[pl.ds(row0, RB)], acc.at[pl.ds(row0, RB)])
  pltpu.sync_copy(ones_hbm, onesv)
  pltpu.sync_copy(dst_hbm.at[c * NS + s], dstv)
  plsc.subcore_barrier()

  def body(j, carry):
    pltpu.sync_copy(onesv, acc.at[dstv.at[j]], add=True)
    return carry

  lax.fori_loop(0, nch, body, 0)
  plsc.subcore_barrier()
  pltpu.sync_copy(acc.at[pl.ds(row0, RB)], out_hbm.at[c].at[pl.ds(row0, RB)])


@functools.lru_cache(maxsize=None)
def _sc_degree():
  return pl.kernel(
      _sc_degree_body,
      out_type=_f32((NC, NPAD, DEGW)),
      mesh=_sc_mesh(),
      scratch_types=[
          pltpu.VMEM((DROWS // (NC * NS), KD), jnp.int32),
          pltpu.VMEM((KD, DEGW), jnp.float32),
          pltpu.VMEM_SHARED((NPAD, DEGW), jnp.float32),
      ],
      compiler_params=pltpu.CompilerParams(use_tc_tiling_on_sc=False),
  )


def _spmv_pass(gq, outq, src_s, dst_s, srcb, dstb, rowbuf, acc, gsem, ssem,
               row0):
  pltpu.sync_copy(gq.at[pl.ds(row0, RB)], acc.at[pl.ds(row0, RB)])
  plsc.subcore_barrier()

  ng = ECH // NBUF

  def body(i, carry):
    j = NBUF * i
    pltpu.sync_copy(src_s.at[pl.ds(j, NBUF)], srcb)
    pltpu.sync_copy(dst_s.at[pl.ds(j, NBUF)], dstb)
    for b in range(NBUF):
      @pl.when(i > 0)
      def _():
        pltpu.make_async_copy(rowbuf.at[b], acc.at[dstb.at[b]],
                              ssem.at[b]).wait()
      pltpu.async_copy(gq.at[srcb.at[b]], rowbuf.at[b], gsem.at[b])
    for b in range(NBUF):
      pltpu.make_async_copy(gq.at[srcb.at[b]], rowbuf.at[b], gsem.at[b]).wait()
      pltpu.async_copy(rowbuf.at[b], acc.at[dstb.at[b]], ssem.at[b], add=True)
    return carry

  lax.fori_loop(0, ng, body, 0)
  for b in range(NBUF):
    pltpu.make_async_copy(rowbuf.at[b], acc.at[dstb.at[0]], ssem.at[b]).wait()
  plsc.subcore_barrier()
  pltpu.sync_copy(acc.at[pl.ds(row0, RB)], outq.at[pl.ds(row0, RB)])


def _sc_spmv_body(g_hbm, src_hbm, dst_hbm, out_hbm, srcb, dstb, rowbuf, acc,
                  gsem, ssem):
  c = lax.axis_index("c")
  s = lax.axis_index("s")
  row0 = s * RB
  t = c * NS + s
  _spmv_pass(g_hbm, out_hbm.at[c], src_hbm.at[t], dst_hbm.at[t], srcb, dstb,
             rowbuf, acc, gsem, ssem, row0)


@functools.lru_cache(maxsize=None)
def _sc_spmv():
  return pl.kernel(
      _sc_spmv_body,
      out_type=_f32((NC, NPAD, D)),
      mesh=_sc_mesh(),
      scratch_types=[
          pltpu.VMEM((NBUF, K), jnp.int32),
          pltpu.VMEM((NBUF, K), jnp.int32),
          pltpu.VMEM((NBUF, K, D), jnp.float32),
          pltpu.VMEM_SHARED((NPAD, D), jnp.float32),
          pltpu.SemaphoreType.DMA((NBUF,)),
          pltpu.SemaphoreType.DMA((NBUF,)),
      ],
      compiler_params=pltpu.CompilerParams(use_tc_tiling_on_sc=False),
  )


def _tc_prep_body(deg_ref, x_ref, dinv_ref, xs_ref):
  deg = deg_ref[0, :, :1] + deg_ref[1, :, :1] + 1.0
  dinv = lax.rsqrt(jnp.maximum(deg, 1.0))
  dinv_ref[...] = dinv
  xs_ref[...] = dinv * x_ref[...]


def _tc_prep(deg2, x_pad):
  return pl.pallas_call(
      _tc_prep_body,
      grid=(16,),
      in_specs=[
          pl.BlockSpec((NC, RB, DEGW), lambda i: (0, i, 0)),
          pl.BlockSpec((RB, D), lambda i: (i, 0)),
      ],
      out_specs=[
          pl.BlockSpec((RB, 1), lambda i: (i, 0)),
          pl.BlockSpec((RB, D), lambda i: (i, 0)),
      ],
      out_shape=[_f32((NPAD, 1)), _f32((NPAD, D))],
  )(deg2, x_pad)


def _tc_conv1_body(u_ref, xs_ref, dinv_ref, w1_ref, b1_ref, ga_ref, gb_ref):
  dinv = dinv_ref[...]
  t1 = dinv * (u_ref[0] + u_ref[1] - xs_ref[...])
  h = jnp.dot(t1, w1_ref[...], preferred_element_type=jnp.float32)
  g = dinv * jnp.maximum(h + b1_ref[...], 0.0)
  ga_ref[...] = g[:, :D]
  gb_ref[...] = g[:, D:]


def _tc_conv1(u1, xs, dinv, W1, b1):
  return pl.pallas_call(
      _tc_conv1_body,
      grid=(16,),
      in_specs=[
          pl.BlockSpec((NC, RB, D), lambda i: (0, i, 0)),
          pl.BlockSpec((RB, D), lambda i: (i, 0)),
          pl.BlockSpec((RB, 1), lambda i: (i, 0)),
          pl.BlockSpec((D, H), lambda i: (0, 0)),
          pl.BlockSpec((1, H), lambda i: (0, 0)),
      ],
      out_specs=[
          pl.BlockSpec((RB, D), lambda i: (i, 0)),
          pl.BlockSpec((RB, D), lambda i: (i, 0)),
      ],
      out_shape=[_f32((NPAD, D)), _f32((NPAD, D))],
  )(u1, xs, dinv, W1, b1)


def _tc_conv2_body(ua_ref, ub_ref, ga_ref, gb_ref, dinv_ref, w2_ref, b2_ref,
                   wh_ref, bh_ref, out_ref, acc_ref):
  i = pl.program_id(0)
  dinv = dinv_ref[...]
  t2a = dinv * (ua_ref[0] + ua_ref[1] - ga_ref[...])
  t2b = dinv * (ub_ref[0] + ub_ref[1] - gb_ref[...])
  t2 = jnp.concatenate([t2a, t2b], axis=1)
  g2 = jnp.maximum(
      jnp.dot(t2, w2_ref[...], preferred_element_type=jnp.float32)
      + b2_ref[...], 0.0)
  rows = i * RB + lax.broadcasted_iota(jnp.int32, (RB, 1), 0)
  g2 = jnp.where(rows < N, g2, 0.0)
  part = jnp.sum(g2, axis=0, keepdims=True)

  @pl.when(i == 0)
  def _():
    acc_ref[...] = jnp.zeros_like(acc_ref)

  acc_ref[...] += part

  @pl.when(i == 15)
  def _():
    out_ref[...] = (
        jnp.dot(acc_ref[...], wh_ref[...], preferred_element_type=jnp.float32)
        + bh_ref[...])


def _tc_conv2(u2a, u2b, g1sa, g1sb, dinv, W2, b2, Wh, bh):
  return pl.pallas_call(
      _tc_conv2_body,
      grid=(16,),
      in_specs=[
          pl.BlockSpec((NC, RB, D), lambda i: (0, i, 0)),
          pl.BlockSpec((NC, RB, D), lambda i: (0, i, 0)),
          pl.BlockSpec((RB, D), lambda i: (i, 0)),
          pl.BlockSpec((RB, D), lambda i: (i, 0)),
          pl.BlockSpec((RB, 1), lambda i: (i, 0)),
          pl.BlockSpec((H, H), lambda i: (0, 0)),
          pl.BlockSpec((1, H), lambda i: (0, 0)),
          pl.BlockSpec((H, C), lambda i: (0, 0)),
          pl.BlockSpec((1, C), lambda i: (0, 0)),
      ],
      out_specs=pl.BlockSpec((1, C), lambda i: (0, 0)),
      out_shape=_f32((1, C)),
      scratch_shapes=[pltpu.VMEM((1, H), jnp.float32)],
  )(u2a, u2b, g1sa, g1sb, dinv, W2, b2, Wh, bh)


def kernel(x, edge_index, W1, b1, W2, b2, Wh, bh):
  srcd = edge_index[0].reshape(NC * NS, DROWS // (NC * NS), KD)
  dstd = edge_index[1].reshape(NC * NS, DROWS // (NC * NS), KD)
  src_es = edge_index[0].reshape(NC * NS, ECH, K)
  dst_es = edge_index[1].reshape(NC * NS, ECH, K)
  x_pad = jnp.pad(x, ((0, NPAD - N), (0, 0)))
  zeros_deg = jnp.zeros((NPAD, DEGW), jnp.float32)
  ones_deg = jnp.ones((KD, DEGW), jnp.float32)

  deg2 = _sc_degree()(dstd, zeros_deg, ones_deg)
  dinv, xs = _tc_prep(deg2, x_pad)
  u1 = _sc_spmv()(xs, src_es, dst_es)
  g1sa, g1sb = _tc_conv1(u1, xs, dinv, W1, b1.reshape(1, H))
  u2a = _sc_spmv()(g1sa, src_es, dst_es)
  u2b = _sc_spmv()(g1sb, src_es, dst_es)
  return _tc_conv2(u2a, u2b, g1sa, g1sb, dinv, W2, b2.reshape(1, H), Wh,
                   bh.reshape(1, C))


# speedup vs baseline: 1.1825x; 1.1825x over previous
import functools

import jax
import jax.numpy as jnp
from jax import lax
from jax.experimental import pallas as pl
from jax.experimental.pallas import tpu as pltpu
from jax.experimental.pallas import tpu_sc as plsc

N = 10000
E = 320000
D = 128
H = 256
C = 10

NPAD = 10112
RB = NPAD // 16
KD = 80
DROWS = E // KD
K = 40
EROWS = E // K

NC = 2
NS = 16
NCH = EROWS // NS


def _f32(shape):
  return jax.ShapeDtypeStruct(shape, jnp.float32)


@functools.lru_cache(maxsize=None)
def _sc_mesh():
  return plsc.VectorSubcoreMesh(
      core_axis_name="c", subcore_axis_name="s", num_cores=NC, num_subcores=NS)


DEGW = 8


def _sc_degree_body(dst_hbm, zeros_hbm, ones_hbm, out_hbm, dstv, onesv, acc):
  c = lax.axis_index("c")
  s = lax.axis_index("s")
  nch = DROWS // (NC * NS)
  row0 = s * RB
  pltpu.sync_copy(zeros_hbm.at[pl.ds(row0, RB)], acc.at[pl.ds(row0, RB)])
  pltpu.sync_copy(ones_hbm, onesv)
  pltpu.sync_copy(dst_hbm.at[c * NS + s], dstv)
  plsc.subcore_barrier()

  def body(j, carry):
    pltpu.sync_copy(onesv, acc.at[dstv.at[j]], add=True)
    return carry

  lax.fori_loop(0, nch, body, 0)
  plsc.subcore_barrier()
  pltpu.sync_copy(acc.at[pl.ds(row0, RB)], out_hbm.at[c].at[pl.ds(row0, RB)])


@functools.lru_cache(maxsize=None)
def _sc_degree(w=DEGW):
  return pl.kernel(
      _sc_degree_body,
      out_type=_f32((NC, NPAD, w)),
      mesh=_sc_mesh(),
      scratch_types=[
          pltpu.VMEM((DROWS // (NC * NS), KD), jnp.int32),
          pltpu.VMEM((KD, w), jnp.float32),
          pltpu.VMEM_SHARED((NPAD, w), jnp.float32),
      ],
      compiler_params=pltpu.CompilerParams(use_tc_tiling_on_sc=False),
  )


NBUF = 10


def _spmv_pass(gq, outq, srcv, dstv, rowbuf, acc, gsem, ssem, row0):
  pltpu.sync_copy(gq.at[pl.ds(row0, RB)], acc.at[pl.ds(row0, RB)])
  plsc.subcore_barrier()

  ng = NCH // NBUF

  def body(i, carry):
    j = NBUF * i
    for b in range(NBUF):
      @pl.when(i > 0)
      def _():
        pltpu.make_async_copy(rowbuf.at[b], acc.at[dstv.at[j + b]],
                              ssem.at[b]).wait()
      pltpu.async_copy(gq.at[srcv.at[j + b]], rowbuf.at[b], gsem.at[b])
    for b in range(NBUF):
      pltpu.make_async_copy(gq.at[srcv.at[j + b]], rowbuf.at[b],
                            gsem.at[b]).wait()
      pltpu.async_copy(rowbuf.at[b], acc.at[dstv.at[j + b]], ssem.at[b],
                       add=True)
    return carry

  lax.fori_loop(0, ng, body, 0)
  for b in range(NBUF):
    pltpu.make_async_copy(rowbuf.at[b], acc.at[dstv.at[0]], ssem.at[b]).wait()
  plsc.subcore_barrier()
  pltpu.sync_copy(acc.at[pl.ds(row0, RB)], outq.at[pl.ds(row0, RB)])


def _sc_spmv_body(g_hbm, src_hbm, dst_hbm, out_hbm, srcv, dstv, rowbuf, acc,
                  gsem, ssem):
  c = lax.axis_index("c")
  s = lax.axis_index("s")
  row0 = s * RB
  pltpu.sync_copy(src_hbm.at[s], srcv)
  pltpu.sync_copy(dst_hbm.at[s], dstv)
  _spmv_pass(g_hbm.at[c], out_hbm.at[c], srcv, dstv, rowbuf, acc, gsem, ssem,
             row0)


@functools.lru_cache(maxsize=None)
def _sc_spmv(w):
  return pl.kernel(
      _sc_spmv_body,
      out_type=_f32((NC, NPAD, w)),
      mesh=_sc_mesh(),
      scratch_types=[
          pltpu.VMEM((NCH, K), jnp.int32),
          pltpu.VMEM((NCH, K), jnp.int32),
          pltpu.VMEM((NBUF, K, w), jnp.float32),
          pltpu.VMEM_SHARED((NPAD, w), jnp.float32),
          pltpu.SemaphoreType.DMA((NBUF,)),
          pltpu.SemaphoreType.DMA((NBUF,)),
      ],
      compiler_params=pltpu.CompilerParams(use_tc_tiling_on_sc=False),
  )


def _sc_spmv4_body(g_hbm, src_hbm, dst_hbm, out_hbm, srcv, dstv, rowbuf, acc,
                   gsem, ssem):
  c = lax.axis_index("c")
  s = lax.axis_index("s")
  row0 = s * RB
  pltpu.sync_copy(src_hbm.at[s], srcv)
  pltpu.sync_copy(dst_hbm.at[s], dstv)
  for q in range(2):
    qq = 2 * q + c
    _spmv_pass(g_hbm.at[qq], out_hbm.at[qq], srcv, dstv, rowbuf, acc, gsem,
               ssem, row0)


@functools.lru_cache(maxsize=None)
def _sc_spmv4(w):
  return pl.kernel(
      _sc_spmv4_body,
      out_type=_f32((4, NPAD, w)),
      mesh=_sc_mesh(),
      scratch_types=[
          pltpu.VMEM((NCH, K), jnp.int32),
          pltpu.VMEM((NCH, K), jnp.int32),
          pltpu.VMEM((NBUF, K, w), jnp.float32),
          pltpu.VMEM_SHARED((NPAD, w), jnp.float32),
          pltpu.SemaphoreType.DMA((NBUF,)),
          pltpu.SemaphoreType.DMA((NBUF,)),
      ],
      compiler_params=pltpu.CompilerParams(use_tc_tiling_on_sc=False),
  )


def _tc_prep_body(deg_ref, x_ref, dinv_ref, xs_ref):
  deg = deg_ref[0, :, :1] + deg_ref[1, :, :1] + 1.0
  dinv = lax.rsqrt(jnp.maximum(deg, 1.0))
  dinv_ref[...] = dinv
  xs = dinv * x_ref[...]
  xs_ref[0] = xs[:, : D // 2]
  xs_ref[1] = xs[:, D // 2 :]


def _tc_prep(deg2, x_pad):
  return pl.pallas_call(
      _tc_prep_body,
      grid=(16,),
      in_specs=[
          pl.BlockSpec((NC, RB, DEGW), lambda i: (0, i, 0)),
          pl.BlockSpec((RB, D), lambda i: (i, 0)),
      ],
      out_specs=[
          pl.BlockSpec((RB, 1), lambda i: (i, 0)),
          pl.BlockSpec((NC, RB, D // 2), lambda i: (0, i, 0)),
      ],
      out_shape=[_f32((NPAD, 1)), _f32((NC, NPAD, D // 2))],
  )(deg2, x_pad)


def _tc_conv1_body(u_ref, dinv_ref, w1_ref, b1_ref, g_ref):
  dinv = dinv_ref[...]
  t1 = dinv * jnp.concatenate([u_ref[0], u_ref[1]], axis=1)
  h = jnp.dot(t1, w1_ref[...], preferred_element_type=jnp.float32)
  g = dinv * jnp.maximum(h + b1_ref[...], 0.0)
  g_ref[0] = g[:, : D // 2]
  g_ref[1] = g[:, D // 2 : D]
  g_ref[2] = g[:, D : 3 * D // 2]
  g_ref[3] = g[:, 3 * D // 2 :]


def _tc_conv1(u1, dinv, W1, b1):
  return pl.pallas_call(
      _tc_conv1_body,
      grid=(16,),
      in_specs=[
          pl.BlockSpec((NC, RB, D // 2), lambda i: (0, i, 0)),
          pl.BlockSpec((RB, 1), lambda i: (i, 0)),
          pl.BlockSpec((D, H), lambda i: (0, 0)),
          pl.BlockSpec((1, H), lambda i: (0, 0)),
      ],
      out_specs=pl.BlockSpec((4, RB, D // 2), lambda i: (0, i, 0)),
      out_shape=_f32((4, NPAD, D // 2)),
  )(u1, dinv, W1, b1)


def _tc_conv2_body(u_ref, dinv_ref, w2_ref, b2_ref, wh_ref, bh_ref,
                   out_ref, acc_ref):
  i = pl.program_id(0)
  dinv = dinv_ref[...]
  t2 = dinv * jnp.concatenate(
      [u_ref[0], u_ref[1], u_ref[2], u_ref[3]], axis=1)
  g2 = jnp.maximum(
      jnp.dot(t2, w2_ref[...], preferred_element_type=jnp.float32)
      + b2_ref[...], 0.0)
  rows = i * RB + lax.broadcasted_iota(jnp.int32, (RB, 1), 0)
  g2 = jnp.where(rows < N, g2, 0.0)
  part = jnp.sum(g2, axis=0, keepdims=True)

  @pl.when(i == 0)
  def _():
    acc_ref[...] = jnp.zeros_like(acc_ref)

  acc_ref[...] += part

  @pl.when(i == 15)
  def _():
    out_ref[...] = (
        jnp.dot(acc_ref[...], wh_ref[...], preferred_element_type=jnp.float32)
        + bh_ref[...])


def _tc_conv2(u2, dinv, W2, b2, Wh, bh):
  return pl.pallas_call(
      _tc_conv2_body,
      grid=(16,),
      in_specs=[
          pl.BlockSpec((4, RB, D // 2), lambda i: (0, i, 0)),
          pl.BlockSpec((RB, 1), lambda i: (i, 0)),
          pl.BlockSpec((H, H), lambda i: (0, 0)),
          pl.BlockSpec((1, H), lambda i: (0, 0)),
          pl.BlockSpec((H, C), lambda i: (0, 0)),
          pl.BlockSpec((1, C), lambda i: (0, 0)),
      ],
      out_specs=pl.BlockSpec((1, C), lambda i: (0, 0)),
      out_shape=_f32((1, C)),
      scratch_shapes=[pltpu.VMEM((1, H), jnp.float32)],
  )(u2, dinv, W2, b2, Wh, bh)


def kernel(x, edge_index, W1, b1, W2, b2, Wh, bh):
  dst1 = edge_index[1].reshape(NC * NS, DROWS // (NC * NS), KD)
  src2 = edge_index[0].reshape(NS, NCH, K)
  dst2 = edge_index[1].reshape(NS, NCH, K)
  x_pad = jnp.pad(x, ((0, NPAD - N), (0, 0)))
  zeros_col = jnp.zeros((NPAD, DEGW), jnp.float32)
  ones_col = jnp.ones((KD, DEGW), jnp.float32)

  deg2 = _sc_degree()(dst1, zeros_col, ones_col)
  dinv, xs2 = _tc_prep(deg2, x_pad)
  u1 = _sc_spmv(D // 2)(xs2, src2, dst2)
  g1s4 = _tc_conv1(u1, dinv, W1, b1.reshape(1, H))
  u2 = _sc_spmv4(D // 2)(g1s4, src2, dst2)
  return _tc_conv2(u2, dinv, W2, b2.reshape(1, H), Wh, bh.reshape(1, C))


# speedup vs baseline: 1.2431x; 1.0512x over previous
import functools

import jax
import jax.numpy as jnp
from jax import lax
from jax.experimental import pallas as pl
from jax.experimental.pallas import tpu as pltpu
from jax.experimental.pallas import tpu_sc as plsc

N = 10000
E = 320000
D = 128
H = 256
C = 10

NPAD = 10112
RB = NPAD // 16
KD = 80
DROWS = E // KD
K = 80
EROWS = E // K

NC = 2
NS = 16
NCH = EROWS // NS


def _f32(shape):
  return jax.ShapeDtypeStruct(shape, jnp.float32)


@functools.lru_cache(maxsize=None)
def _sc_mesh():
  return plsc.VectorSubcoreMesh(
      core_axis_name="c", subcore_axis_name="s", num_cores=NC, num_subcores=NS)


DEGW = 8


def _sc_degree_body(dst_hbm, zeros_hbm, ones_hbm, out_hbm, dstv, onesv, acc):
  c = lax.axis_index("c")
  s = lax.axis_index("s")
  nch = DROWS // (NC * NS)
  row0 = s * RB
  pltpu.sync_copy(zeros_hbm.at[pl.ds(row0, RB)], acc.at[pl.ds(row0, RB)])
  pltpu.sync_copy(ones_hbm, onesv)
  pltpu.sync_copy(dst_hbm.at[c * NS + s], dstv)
  plsc.subcore_barrier()

  def body(j, carry):
    pltpu.sync_copy(onesv, acc.at[dstv.at[j]], add=True)
    return carry

  lax.fori_loop(0, nch, body, 0)
  plsc.subcore_barrier()
  pltpu.sync_copy(acc.at[pl.ds(row0, RB)], out_hbm.at[c].at[pl.ds(row0, RB)])


@functools.lru_cache(maxsize=None)
def _sc_degree(w=DEGW):
  return pl.kernel(
      _sc_degree_body,
      out_type=_f32((NC, NPAD, w)),
      mesh=_sc_mesh(),
      scratch_types=[
          pltpu.VMEM((DROWS // (NC * NS), KD), jnp.int32),
          pltpu.VMEM((KD, w), jnp.float32),
          pltpu.VMEM_SHARED((NPAD, w), jnp.float32),
      ],
      compiler_params=pltpu.CompilerParams(use_tc_tiling_on_sc=False),
  )


NBUF = 5


def _spmv_pass(gq, outq, srcv, dstv, rowbuf, acc, gsem, ssem, row0):
  pltpu.sync_copy(gq.at[pl.ds(row0, RB)], acc.at[pl.ds(row0, RB)])
  plsc.subcore_barrier()

  ng = NCH // NBUF

  def body(i, carry):
    j = NBUF * i
    for b in range(NBUF):
      @pl.when(i > 0)
      def _():
        pltpu.make_async_copy(rowbuf.at[b], acc.at[dstv.at[j + b]],
                              ssem.at[b]).wait()
      pltpu.async_copy(gq.at[srcv.at[j + b]], rowbuf.at[b], gsem.at[b])
    for b in range(NBUF):
      pltpu.make_async_copy(gq.at[srcv.at[j + b]], rowbuf.at[b],
                            gsem.at[b]).wait()
      pltpu.async_copy(rowbuf.at[b], acc.at[dstv.at[j + b]], ssem.at[b],
                       add=True)
    return carry

  lax.fori_loop(0, ng, body, 0)
  for b in range(NBUF):
    pltpu.make_async_copy(rowbuf.at[b], acc.at[dstv.at[0]], ssem.at[b]).wait()
  plsc.subcore_barrier()
  pltpu.sync_copy(acc.at[pl.ds(row0, RB)], outq.at[pl.ds(row0, RB)])


def _sc_spmv_body(g_hbm, src_hbm, dst_hbm, out_hbm, srcv, dstv, rowbuf, acc,
                  gsem, ssem):
  c = lax.axis_index("c")
  s = lax.axis_index("s")
  row0 = s * RB
  pltpu.sync_copy(src_hbm.at[s], srcv)
  pltpu.sync_copy(dst_hbm.at[s], dstv)
  _spmv_pass(g_hbm.at[c], out_hbm.at[c], srcv, dstv, rowbuf, acc, gsem, ssem,
             row0)


@functools.lru_cache(maxsize=None)
def _sc_spmv(w):
  return pl.kernel(
      _sc_spmv_body,
      out_type=_f32((NC, NPAD, w)),
      mesh=_sc_mesh(),
      scratch_types=[
          pltpu.VMEM((NCH, K), jnp.int32),
          pltpu.VMEM((NCH, K), jnp.int32),
          pltpu.VMEM((NBUF, K, w), jnp.float32),
          pltpu.VMEM_SHARED((NPAD, w), jnp.float32),
          pltpu.SemaphoreType.DMA((NBUF,)),
          pltpu.SemaphoreType.DMA((NBUF,)),
      ],
      compiler_params=pltpu.CompilerParams(use_tc_tiling_on_sc=False),
  )


def _sc_spmv4_body(g_hbm, src_hbm, dst_hbm, out_hbm, srcv, dstv, rowbuf, acc,
                   gsem, ssem):
  c = lax.axis_index("c")
  s = lax.axis_index("s")
  row0 = s * RB
  pltpu.sync_copy(src_hbm.at[s], srcv)
  pltpu.sync_copy(dst_hbm.at[s], dstv)
  for q in range(2):
    qq = 2 * q + c
    _spmv_pass(g_hbm.at[qq], out_hbm.at[qq], srcv, dstv, rowbuf, acc, gsem,
               ssem, row0)


@functools.lru_cache(maxsize=None)
def _sc_spmv4(w):
  return pl.kernel(
      _sc_spmv4_body,
      out_type=_f32((4, NPAD, w)),
      mesh=_sc_mesh(),
      scratch_types=[
          pltpu.VMEM((NCH, K), jnp.int32),
          pltpu.VMEM((NCH, K), jnp.int32),
          pltpu.VMEM((NBUF, K, w), jnp.float32),
          pltpu.VMEM_SHARED((NPAD, w), jnp.float32),
          pltpu.SemaphoreType.DMA((NBUF,)),
          pltpu.SemaphoreType.DMA((NBUF,)),
      ],
      compiler_params=pltpu.CompilerParams(use_tc_tiling_on_sc=False),
  )


def _tc_prep_body(deg_ref, x_ref, dinv_ref, xs_ref):
  deg = deg_ref[0, :, :1] + deg_ref[1, :, :1] + 1.0
  dinv = lax.rsqrt(jnp.maximum(deg, 1.0))
  dinv_ref[...] = dinv
  xs = dinv * x_ref[...]
  xs_ref[0] = xs[:, : D // 2]
  xs_ref[1] = xs[:, D // 2 :]


def _tc_prep(deg2, x_pad):
  return pl.pallas_call(
      _tc_prep_body,
      grid=(16,),
      in_specs=[
          pl.BlockSpec((NC, RB, DEGW), lambda i: (0, i, 0)),
          pl.BlockSpec((RB, D), lambda i: (i, 0)),
      ],
      out_specs=[
          pl.BlockSpec((RB, 1), lambda i: (i, 0)),
          pl.BlockSpec((NC, RB, D // 2), lambda i: (0, i, 0)),
      ],
      out_shape=[_f32((NPAD, 1)), _f32((NC, NPAD, D // 2))],
  )(deg2, x_pad)


def _tc_conv1_body(u_ref, dinv_ref, w1_ref, b1_ref, g_ref):
  dinv = dinv_ref[...]
  t1 = dinv * jnp.concatenate([u_ref[0], u_ref[1]], axis=1)
  h = jnp.dot(t1, w1_ref[...], preferred_element_type=jnp.float32)
  g = dinv * jnp.maximum(h + b1_ref[...], 0.0)
  g_ref[0] = g[:, : D // 2]
  g_ref[1] = g[:, D // 2 : D]
  g_ref[2] = g[:, D : 3 * D // 2]
  g_ref[3] = g[:, 3 * D // 2 :]


def _tc_conv1(u1, dinv, W1, b1):
  return pl.pallas_call(
      _tc_conv1_body,
      grid=(16,),
      in_specs=[
          pl.BlockSpec((NC, RB, D // 2), lambda i: (0, i, 0)),
          pl.BlockSpec((RB, 1), lambda i: (i, 0)),
          pl.BlockSpec((D, H), lambda i: (0, 0)),
          pl.BlockSpec((1, H), lambda i: (0, 0)),
      ],
      out_specs=pl.BlockSpec((4, RB, D // 2), lambda i: (0, i, 0)),
      out_shape=_f32((4, NPAD, D // 2)),
  )(u1, dinv, W1, b1)


def _tc_conv2_body(u_ref, dinv_ref, w2_ref, b2_ref, wh_ref, bh_ref,
                   out_ref, acc_ref):
  i = pl.program_id(0)
  dinv = dinv_ref[...]
  t2 = dinv * jnp.concatenate(
      [u_ref[0], u_ref[1], u_ref[2], u_ref[3]], axis=1)
  g2 = jnp.maximum(
      jnp.dot(t2, w2_ref[...], preferred_element_type=jnp.float32)
      + b2_ref[...], 0.0)
  rows = i * RB + lax.broadcasted_iota(jnp.int32, (RB, 1), 0)
  g2 = jnp.where(rows < N, g2, 0.0)
  part = jnp.sum(g2, axis=0, keepdims=True)

  @pl.when(i == 0)
  def _():
    acc_ref[...] = jnp.zeros_like(acc_ref)

  acc_ref[...] += part

  @pl.when(i == 15)
  def _():
    out_ref[...] = (
        jnp.dot(acc_ref[...], wh_ref[...], preferred_element_type=jnp.float32)
        + bh_ref[...])


def _tc_conv2(u2, dinv, W2, b2, Wh, bh):
  return pl.pallas_call(
      _tc_conv2_body,
      grid=(16,),
      in_specs=[
          pl.BlockSpec((4, RB, D // 2), lambda i: (0, i, 0)),
          pl.BlockSpec((RB, 1), lambda i: (i, 0)),
          pl.BlockSpec((H, H), lambda i: (0, 0)),
          pl.BlockSpec((1, H), lambda i: (0, 0)),
          pl.BlockSpec((H, C), lambda i: (0, 0)),
          pl.BlockSpec((1, C), lambda i: (0, 0)),
      ],
      out_specs=pl.BlockSpec((1, C), lambda i: (0, 0)),
      out_shape=_f32((1, C)),
      scratch_shapes=[pltpu.VMEM((1, H), jnp.float32)],
  )(u2, dinv, W2, b2, Wh, bh)


def kernel(x, edge_index, W1, b1, W2, b2, Wh, bh):
  dst1 = edge_index[1].reshape(NC * NS, DROWS // (NC * NS), KD)
  src2 = edge_index[0].reshape(NS, NCH, K)
  dst2 = edge_index[1].reshape(NS, NCH, K)
  x_pad = jnp.pad(x, ((0, NPAD - N), (0, 0)))
  zeros_col = jnp.zeros((NPAD, DEGW), jnp.float32)
  ones_col = jnp.ones((KD, DEGW), jnp.float32)

  deg2 = _sc_degree()(dst1, zeros_col, ones_col)
  dinv, xs2 = _tc_prep(deg2, x_pad)
  u1 = _sc_spmv(D // 2)(xs2, src2, dst2)
  g1s4 = _tc_conv1(u1, dinv, W1, b1.reshape(1, H))
  u2 = _sc_spmv4(D // 2)(g1s4, src2, dst2)
  return _tc_conv2(u2, dinv, W2, b2.reshape(1, H), Wh, bh.reshape(1, C))


# speedup vs baseline: 1.2598x; 1.0134x over previous
import functools

import jax
import jax.numpy as jnp
from jax import lax
from jax.experimental import pallas as pl
from jax.experimental.pallas import tpu as pltpu
from jax.experimental.pallas import tpu_sc as plsc

N = 10000
E = 320000
D = 128
H = 256
C = 10

NPAD = 10112
RB = NPAD // 16
KD = 80
DROWS = E // KD
K = 80
EROWS = E // K

NC = 2
NS = 16
NCH = EROWS // NS


def _f32(shape):
  return jax.ShapeDtypeStruct(shape, jnp.float32)


@functools.lru_cache(maxsize=None)
def _sc_mesh():
  return plsc.VectorSubcoreMesh(
      core_axis_name="c", subcore_axis_name="s", num_cores=NC, num_subcores=NS)


DEGW = 8


def _sc_degree_body(dst_hbm, zeros_hbm, ones_hbm, out_hbm, dstv, onesv, acc,
                    dsem):
  c = lax.axis_index("c")
  s = lax.axis_index("s")
  nch = DROWS // (NC * NS)
  row0 = s * RB
  pltpu.sync_copy(zeros_hbm.at[pl.ds(row0, RB)], acc.at[pl.ds(row0, RB)])
  pltpu.sync_copy(ones_hbm, onesv)
  pltpu.sync_copy(dst_hbm.at[c * NS + s], dstv)
  plsc.subcore_barrier()

  grp = 25

  def body(i, carry):
    j = grp * i
    cps = [
        pltpu.async_copy(onesv, acc.at[dstv.at[j + b]], dsem, add=True)
        for b in range(grp)
    ]
    for cp in cps:
      cp.wait()
    return carry

  lax.fori_loop(0, nch // grp, body, 0)
  plsc.subcore_barrier()
  pltpu.sync_copy(acc.at[pl.ds(row0, RB)], out_hbm.at[c].at[pl.ds(row0, RB)])


@functools.lru_cache(maxsize=None)
def _sc_degree(w=DEGW):
  return pl.kernel(
      _sc_degree_body,
      out_type=_f32((NC, NPAD, w)),
      mesh=_sc_mesh(),
      scratch_types=[
          pltpu.VMEM((DROWS // (NC * NS), KD), jnp.int32),
          pltpu.VMEM((KD, w), jnp.float32),
          pltpu.VMEM_SHARED((NPAD, w), jnp.float32),
          pltpu.SemaphoreType.DMA,
      ],
      compiler_params=pltpu.CompilerParams(use_tc_tiling_on_sc=False),
  )


NBUF = 5


def _spmv_pass(gq, outq, srcv, dstv, rowbuf, acc, gsem, ssem, row0):
  pltpu.sync_copy(gq.at[pl.ds(row0, RB)], acc.at[pl.ds(row0, RB)])
  plsc.subcore_barrier()

  ng = NCH // NBUF

  def body(i, carry):
    j = NBUF * i
    for b in range(NBUF):
      @pl.when(i > 0)
      def _():
        pltpu.make_async_copy(rowbuf.at[b], acc.at[dstv.at[j + b]],
                              ssem.at[b]).wait()
      pltpu.async_copy(gq.at[srcv.at[j + b]], rowbuf.at[b], gsem.at[b])
    for b in range(NBUF):
      pltpu.make_async_copy(gq.at[srcv.at[j + b]], rowbuf.at[b],
                            gsem.at[b]).wait()
      pltpu.async_copy(rowbuf.at[b], acc.at[dstv.at[j + b]], ssem.at[b],
                       add=True)
    return carry

  lax.fori_loop(0, ng, body, 0)
  for b in range(NBUF):
    pltpu.make_async_copy(rowbuf.at[b], acc.at[dstv.at[0]], ssem.at[b]).wait()
  plsc.subcore_barrier()
  pltpu.sync_copy(acc.at[pl.ds(row0, RB)], outq.at[pl.ds(row0, RB)])


def _sc_spmv_body(g_hbm, src_hbm, dst_hbm, out_hbm, srcv, dstv, rowbuf, acc,
                  gsem, ssem):
  c = lax.axis_index("c")
  s = lax.axis_index("s")
  row0 = s * RB
  pltpu.sync_copy(src_hbm.at[s], srcv)
  pltpu.sync_copy(dst_hbm.at[s], dstv)
  _spmv_pass(g_hbm.at[c], out_hbm.at[c], srcv, dstv, rowbuf, acc, gsem, ssem,
             row0)


@functools.lru_cache(maxsize=None)
def _sc_spmv(w):
  return pl.kernel(
      _sc_spmv_body,
      out_type=_f32((NC, NPAD, w)),
      mesh=_sc_mesh(),
      scratch_types=[
          pltpu.VMEM((NCH, K), jnp.int32),
          pltpu.VMEM((NCH, K), jnp.int32),
          pltpu.VMEM((NBUF, K, w), jnp.float32),
          pltpu.VMEM_SHARED((NPAD, w), jnp.float32),
          pltpu.SemaphoreType.DMA((NBUF,)),
          pltpu.SemaphoreType.DMA((NBUF,)),
      ],
      compiler_params=pltpu.CompilerParams(use_tc_tiling_on_sc=False),
  )


def _sc_spmv4_body(g_hbm, src_hbm, dst_hbm, out_hbm, srcv, dstv, rowbuf, acc,
                   gsem, ssem):
  c = lax.axis_index("c")
  s = lax.axis_index("s")
  row0 = s * RB
  pltpu.sync_copy(src_hbm.at[s], srcv)
  pltpu.sync_copy(dst_hbm.at[s], dstv)
  for q in range(2):
    qq = 2 * q + c
    _spmv_pass(g_hbm.at[qq], out_hbm.at[qq], srcv, dstv, rowbuf, acc, gsem,
               ssem, row0)


@functools.lru_cache(maxsize=None)
def _sc_spmv4(w):
  return pl.kernel(
      _sc_spmv4_body,
      out_type=_f32((4, NPAD, w)),
      mesh=_sc_mesh(),
      scratch_types=[
          pltpu.VMEM((NCH, K), jnp.int32),
          pltpu.VMEM((NCH, K), jnp.int32),
          pltpu.VMEM((NBUF, K, w), jnp.float32),
          pltpu.VMEM_SHARED((NPAD, w), jnp.float32),
          pltpu.SemaphoreType.DMA((NBUF,)),
          pltpu.SemaphoreType.DMA((NBUF,)),
      ],
      compiler_params=pltpu.CompilerParams(use_tc_tiling_on_sc=False),
  )


def _tc_prep_body(deg_ref, x_ref, dinv_ref, xs_ref):
  deg = deg_ref[0, :, :1] + deg_ref[1, :, :1] + 1.0
  dinv = lax.rsqrt(jnp.maximum(deg, 1.0))
  dinv_ref[...] = dinv
  xs = dinv * x_ref[...]
  xs_ref[0] = xs[:, : D // 2]
  xs_ref[1] = xs[:, D // 2 :]


def _tc_prep(deg2, x_pad):
  return pl.pallas_call(
      _tc_prep_body,
      grid=(16,),
      in_specs=[
          pl.BlockSpec((NC, RB, DEGW), lambda i: (0, i, 0)),
          pl.BlockSpec((RB, D), lambda i: (i, 0)),
      ],
      out_specs=[
          pl.BlockSpec((RB, 1), lambda i: (i, 0)),
          pl.BlockSpec((NC, RB, D // 2), lambda i: (0, i, 0)),
      ],
      out_shape=[_f32((NPAD, 1)), _f32((NC, NPAD, D // 2))],
  )(deg2, x_pad)


def _tc_conv1_body(u_ref, dinv_ref, w1_ref, b1_ref, g_ref):
  dinv = dinv_ref[...]
  t1 = dinv * jnp.concatenate([u_ref[0], u_ref[1]], axis=1)
  h = jnp.dot(t1, w1_ref[...], preferred_element_type=jnp.float32)
  g = dinv * jnp.maximum(h + b1_ref[...], 0.0)
  g_ref[0] = g[:, : D // 2]
  g_ref[1] = g[:, D // 2 : D]
  g_ref[2] = g[:, D : 3 * D // 2]
  g_ref[3] = g[:, 3 * D // 2 :]


def _tc_conv1(u1, dinv, W1, b1):
  return pl.pallas_call(
      _tc_conv1_body,
      grid=(16,),
      in_specs=[
          pl.BlockSpec((NC, RB, D // 2), lambda i: (0, i, 0)),
          pl.BlockSpec((RB, 1), lambda i: (i, 0)),
          pl.BlockSpec((D, H), lambda i: (0, 0)),
          pl.BlockSpec((1, H), lambda i: (0, 0)),
      ],
      out_specs=pl.BlockSpec((4, RB, D // 2), lambda i: (0, i, 0)),
      out_shape=_f32((4, NPAD, D // 2)),
  )(u1, dinv, W1, b1)


def _tc_conv2_body(u_ref, dinv_ref, w2_ref, b2_ref, wh_ref, bh_ref,
                   out_ref, acc_ref):
  i = pl.program_id(0)
  dinv = dinv_ref[...]
  t2 = dinv * jnp.concatenate(
      [u_ref[0], u_ref[1], u_ref[2], u_ref[3]], axis=1)
  g2 = jnp.maximum(
      jnp.dot(t2, w2_ref[...], preferred_element_type=jnp.float32)
      + b2_ref[...], 0.0)
  rows = i * RB + lax.broadcasted_iota(jnp.int32, (RB, 1), 0)
  g2 = jnp.where(rows < N, g2, 0.0)
  part = jnp.sum(g2, axis=0, keepdims=True)

  @pl.when(i == 0)
  def _():
    acc_ref[...] = jnp.zeros_like(acc_ref)

  acc_ref[...] += part

  @pl.when(i == 15)
  def _():
    out_ref[...] = (
        jnp.dot(acc_ref[...], wh_ref[...], preferred_element_type=jnp.float32)
        + bh_ref[...])


def _tc_conv2(u2, dinv, W2, b2, Wh, bh):
  return pl.pallas_call(
      _tc_conv2_body,
      grid=(16,),
      in_specs=[
          pl.BlockSpec((4, RB, D // 2), lambda i: (0, i, 0)),
          pl.BlockSpec((RB, 1), lambda i: (i, 0)),
          pl.BlockSpec((H, H), lambda i: (0, 0)),
          pl.BlockSpec((1, H), lambda i: (0, 0)),
          pl.BlockSpec((H, C), lambda i: (0, 0)),
          pl.BlockSpec((1, C), lambda i: (0, 0)),
      ],
      out_specs=pl.BlockSpec((1, C), lambda i: (0, 0)),
      out_shape=_f32((1, C)),
      scratch_shapes=[pltpu.VMEM((1, H), jnp.float32)],
  )(u2, dinv, W2, b2, Wh, bh)


def kernel(x, edge_index, W1, b1, W2, b2, Wh, bh):
  dst1 = edge_index[1].reshape(NC * NS, DROWS // (NC * NS), KD)
  src2 = edge_index[0].reshape(NS, NCH, K)
  dst2 = edge_index[1].reshape(NS, NCH, K)
  x_pad = jnp.pad(x, ((0, NPAD - N), (0, 0)))
  zeros_col = jnp.zeros((NPAD, DEGW), jnp.float32)
  ones_col = jnp.ones((KD, DEGW), jnp.float32)

  deg2 = _sc_degree()(dst1, zeros_col, ones_col)
  dinv, xs2 = _tc_prep(deg2, x_pad)
  u1 = _sc_spmv(D // 2)(xs2, src2, dst2)
  g1s4 = _tc_conv1(u1, dinv, W1, b1.reshape(1, H))
  u2 = _sc_spmv4(D // 2)(g1s4, src2, dst2)
  return _tc_conv2(u2, dinv, W2, b2.reshape(1, H), Wh, bh.reshape(1, C))
